# baseline (speedup 1.0000x reference)
import numpy as np
import jax
import jax.numpy as jnp
from jax.experimental import pallas as pl
from jax.experimental.pallas import tpu as pltpu

_MAX_DIST = 18
_BOARD = 19
_N = _BOARD * _BOARD


def _build_code_map():
    dirs = [(-1, 0), (1, 0), (0, -1), (0, 1),
            (-1, -1), (-1, 1), (1, -1), (1, 1)]
    tm = np.zeros((_N, _N), np.int8)
    rr, cc = np.meshgrid(np.arange(_BOARD), np.arange(_BOARD), indexing="ij")
    p_flat = rr * _BOARD + cc
    for d, (dr, dc) in enumerate(dirs):
        off = 0 if d < 4 else _MAX_DIST
        for t in range(1, _MAX_DIST + 1):
            tr = rr + dr * t
            tc = cc + dc * t
            valid = (tr >= 0) & (tr < _BOARD) & (tc >= 0) & (tc < _BOARD)
            p = p_flat[valid]
            q = (tr * _BOARD + tc)[valid]
            tm[q, p] = off + t
    return tm


_TM_NP = _build_code_map()


def _body(w_ref, tm_ref, x_hbm_ref, out_ref, x_ref, sem):
    cp = pltpu.make_async_copy(x_hbm_ref, x_ref, sem)
    cp.start()
    tm = tm_ref[...].astype(jnp.bfloat16)
    mt = jnp.zeros((_N, _N), jnp.bfloat16)
    for t in range(1, _MAX_DIST + 1):
        mt = jnp.where(tm == t, w_ref[0, t - 1].astype(jnp.bfloat16), mt)
        mt = jnp.where(tm == _MAX_DIST + t,
                       w_ref[1, t - 1].astype(jnp.bfloat16), mt)
    cp.wait()
    out_ref[...] = jnp.dot(x_ref[...], mt.astype(jnp.float32),
                           preferred_element_type=jnp.float32)


def kernel(x, weight):
    B, C, H, W = x.shape
    xf = x.reshape(B * C, H * W)
    out = pl.pallas_call(
        _body,
        out_shape=jax.ShapeDtypeStruct((B * C, H * W), jnp.float32),
        in_specs=[
            pl.BlockSpec(memory_space=pltpu.SMEM),
            pl.BlockSpec(memory_space=pltpu.VMEM),
            pl.BlockSpec(memory_space=pl.ANY),
        ],
        out_specs=pl.BlockSpec(memory_space=pltpu.VMEM),
        scratch_shapes=[
            pltpu.VMEM((B * C, H * W), jnp.float32),
            pltpu.SemaphoreType.DMA,
        ],
    )(weight, jnp.asarray(_TM_NP), xf)
    return out.reshape(B, C, H, W)


# speedup vs baseline: 1.0068x; 1.0068x over previous
import numpy as np
import jax
import jax.numpy as jnp
from jax.experimental import pallas as pl
from jax.experimental.pallas import tpu as pltpu

_MAX_DIST = 18
_BOARD = 19
_N = _BOARD * _BOARD


def _build_code_map():
    dirs = [(-1, 0), (1, 0), (0, -1), (0, 1),
            (-1, -1), (-1, 1), (1, -1), (1, 1)]
    tm = np.zeros((_N, _N), np.float32)
    rr, cc = np.meshgrid(np.arange(_BOARD), np.arange(_BOARD), indexing="ij")
    p_flat = rr * _BOARD + cc
    for d, (dr, dc) in enumerate(dirs):
        off = 0 if d < 4 else _MAX_DIST
        for t in range(1, _MAX_DIST + 1):
            tr = rr + dr * t
            tc = cc + dc * t
            valid = (tr >= 0) & (tr < _BOARD) & (tc >= 0) & (tc < _BOARD)
            p = p_flat[valid]
            q = (tr * _BOARD + tc)[valid]
            tm[q, p] = off + t
    return tm


_TM_NP = _build_code_map().astype(jnp.bfloat16)


def _body(w_ref, tm_ref, x_ref, out_ref):
    tm = tm_ref[...]
    mt = jnp.zeros((_N, _N), jnp.bfloat16)
    for t in range(1, _MAX_DIST + 1):
        mt = jnp.where(tm == t, w_ref[0, t - 1].astype(jnp.bfloat16), mt)
        mt = jnp.where(tm == _MAX_DIST + t,
                       w_ref[1, t - 1].astype(jnp.bfloat16), mt)
    out_ref[...] = jnp.dot(x_ref[...].astype(jnp.bfloat16), mt,
                           preferred_element_type=jnp.float32)


def kernel(x, weight):
    B, C, H, W = x.shape
    xf = x.reshape(B * C, H * W)
    out = pl.pallas_call(
        _body,
        out_shape=jax.ShapeDtypeStruct((B * C, H * W), jnp.float32),
        in_specs=[
            pl.BlockSpec(memory_space=pltpu.SMEM),
            pl.BlockSpec(memory_space=pltpu.VMEM),
            pl.BlockSpec(memory_space=pltpu.VMEM),
        ],
        out_specs=pl.BlockSpec(memory_space=pltpu.VMEM),
    )(weight, jnp.asarray(_TM_NP), xf)
    return out.reshape(B, C, H, W)


# speedup vs baseline: 1.0147x; 1.0078x over previous
import numpy as np
import jax
import jax.numpy as jnp
from jax.experimental import pallas as pl
from jax.experimental.pallas import tpu as pltpu

_MAX_DIST = 18
_BOARD = 19
_N = _BOARD * _BOARD


def _build_code_map():
    dirs = [(-1, 0), (1, 0), (0, -1), (0, 1),
            (-1, -1), (-1, 1), (1, -1), (1, 1)]
    tm = np.zeros((_N, _N), np.int8)
    rr, cc = np.meshgrid(np.arange(_BOARD), np.arange(_BOARD), indexing="ij")
    p_flat = rr * _BOARD + cc
    for d, (dr, dc) in enumerate(dirs):
        off = 0 if d < 4 else _MAX_DIST
        for t in range(1, _MAX_DIST + 1):
            tr = rr + dr * t
            tc = cc + dc * t
            valid = (tr >= 0) & (tr < _BOARD) & (tc >= 0) & (tc < _BOARD)
            p = p_flat[valid]
            q = (tr * _BOARD + tc)[valid]
            tm[q, p] = off + t
    return tm


_TM_NP = _build_code_map()


def _body(w_ref, tm_ref, x_ref, out_hbm_ref, y_ref, sem0, sem1):
    tm = tm_ref[...].astype(jnp.bfloat16)
    mt = jnp.zeros((_N, _N), jnp.bfloat16)
    for t in range(1, _MAX_DIST + 1):
        mt = jnp.where(tm == t, w_ref[0, t - 1].astype(jnp.bfloat16), mt)
        mt = jnp.where(tm == _MAX_DIST + t,
                       w_ref[1, t - 1].astype(jnp.bfloat16), mt)
    mtf = mt.astype(jnp.float32)
    h = 512
    y_ref[0:h, :] = jnp.dot(x_ref[0:h, :], mtf,
                            preferred_element_type=jnp.float32)
    cp0 = pltpu.make_async_copy(y_ref.at[pl.ds(0, h)],
                                out_hbm_ref.at[pl.ds(0, h)], sem0)
    cp0.start()
    y_ref[h:, :] = jnp.dot(x_ref[h:, :], mtf,
                           preferred_element_type=jnp.float32)
    cp1 = pltpu.make_async_copy(y_ref.at[pl.ds(h, h)],
                                out_hbm_ref.at[pl.ds(h, h)], sem1)
    cp1.start()
    cp0.wait()
    cp1.wait()


def kernel(x, weight):
    B, C, H, W = x.shape
    xf = x.reshape(B * C, H * W)
    out = pl.pallas_call(
        _body,
        out_shape=jax.ShapeDtypeStruct((B * C, H * W), jnp.float32),
        in_specs=[
            pl.BlockSpec(memory_space=pltpu.SMEM),
            pl.BlockSpec(memory_space=pltpu.VMEM),
            pl.BlockSpec(memory_space=pltpu.VMEM),
        ],
        out_specs=pl.BlockSpec(memory_space=pl.ANY),
        scratch_shapes=[
            pltpu.VMEM((B * C, H * W), jnp.float32),
            pltpu.SemaphoreType.DMA,
            pltpu.SemaphoreType.DMA,
        ],
    )(weight, jnp.asarray(_TM_NP), xf)
    return out.reshape(B, C, H, W)


# speedup vs baseline: 1.0166x; 1.0019x over previous
import numpy as np
import jax
import jax.numpy as jnp
from jax.experimental import pallas as pl
from jax.experimental.pallas import tpu as pltpu

_MAX_DIST = 18
_BOARD = 19
_N = _BOARD * _BOARD


def _build_code_map():
    dirs = [(-1, 0), (1, 0), (0, -1), (0, 1),
            (-1, -1), (-1, 1), (1, -1), (1, 1)]
    tm = np.zeros((_N, _N), np.int8)
    rr, cc = np.meshgrid(np.arange(_BOARD), np.arange(_BOARD), indexing="ij")
    p_flat = rr * _BOARD + cc
    for d, (dr, dc) in enumerate(dirs):
        off = 0 if d < 4 else _MAX_DIST
        for t in range(1, _MAX_DIST + 1):
            tr = rr + dr * t
            tc = cc + dc * t
            valid = (tr >= 0) & (tr < _BOARD) & (tc >= 0) & (tc < _BOARD)
            p = p_flat[valid]
            q = (tr * _BOARD + tc)[valid]
            tm[q, p] = off + t
    return tm


_TM_NP = _build_code_map()


def _body(w_ref, tm_ref, x_ref, out_ref):
    tm = tm_ref[...].astype(jnp.bfloat16)
    mt = jnp.zeros((_N, _N), jnp.bfloat16)
    for t in range(1, _MAX_DIST + 1):
        mt = jnp.where(tm == t, w_ref[0, t - 1].astype(jnp.bfloat16), mt)
        mt = jnp.where(tm == _MAX_DIST + t,
                       w_ref[1, t - 1].astype(jnp.bfloat16), mt)
    out_ref[...] = jnp.dot(x_ref[...], mt.astype(jnp.float32),
                           preferred_element_type=jnp.float32)


def kernel(x, weight):
    B, C, H, W = x.shape
    xf = x.reshape(B * C, H * W)
    out = pl.pallas_call(
        _body,
        out_shape=jax.ShapeDtypeStruct((B * C, H * W), jnp.float32),
        in_specs=[
            pl.BlockSpec(memory_space=pltpu.SMEM),
            pl.BlockSpec(memory_space=pltpu.VMEM),
            pl.BlockSpec(memory_space=pltpu.VMEM),
        ],
        out_specs=pl.BlockSpec(memory_space=pltpu.VMEM),
    )(weight, jnp.asarray(_TM_NP), xf)
    return out.reshape(B, C, H, W)


# speedup vs baseline: 1.0401x; 1.0231x over previous
import numpy as np
import jax
import jax.numpy as jnp
from jax.experimental import pallas as pl
from jax.experimental.pallas import tpu as pltpu

_MAX_DIST = 18
_BOARD = 19
_N = _BOARD * _BOARD


def _build_code_map():
    dirs = [(-1, 0), (1, 0), (0, -1), (0, 1),
            (-1, -1), (-1, 1), (1, -1), (1, 1)]
    tm = np.zeros((_N, _N), np.int8)
    rr, cc = np.meshgrid(np.arange(_BOARD), np.arange(_BOARD), indexing="ij")
    p_flat = rr * _BOARD + cc
    for d, (dr, dc) in enumerate(dirs):
        off = 0 if d < 4 else _MAX_DIST
        for t in range(1, _MAX_DIST + 1):
            tr = rr + dr * t
            tc = cc + dc * t
            valid = (tr >= 0) & (tr < _BOARD) & (tc >= 0) & (tc < _BOARD)
            p = p_flat[valid]
            q = (tr * _BOARD + tc)[valid]
            tm[q, p] = off + t
    return tm


_TM_NP = _build_code_map()


def _body(w_ref, tm_ref, x_ref, out_ref, wtab_ref):
    wtab_ref[...] = jnp.zeros((1, 128), jnp.float32)
    wv = w_ref[...]
    wtab_ref[0, pl.ds(1, _MAX_DIST)] = wv[0, :]
    wtab_ref[0, pl.ds(1 + _MAX_DIST, _MAX_DIST)] = wv[1, :]
    wtab = jnp.broadcast_to(wtab_ref[0:1, 0:40], (_N, 40))
    tm32 = tm_ref[...].astype(jnp.int32)
    mt = jnp.take_along_axis(wtab, tm32, axis=1, mode="promise_in_bounds")
    out_ref[...] = jnp.dot(x_ref[...], mt, preferred_element_type=jnp.float32)


def kernel(x, weight):
    B, C, H, W = x.shape
    xf = x.reshape(B * C, H * W)
    out = pl.pallas_call(
        _body,
        out_shape=jax.ShapeDtypeStruct((B * C, H * W), jnp.float32),
        in_specs=[
            pl.BlockSpec(memory_space=pltpu.VMEM),
            pl.BlockSpec(memory_space=pltpu.VMEM),
            pl.BlockSpec(memory_space=pltpu.VMEM),
        ],
        out_specs=pl.BlockSpec(memory_space=pltpu.VMEM),
        scratch_shapes=[pltpu.VMEM((1, 128), jnp.float32)],
    )(weight, jnp.asarray(_TM_NP), xf)
    return out.reshape(B, C, H, W)
